# baseline (speedup 1.0000x reference)
import functools

import jax
import jax.numpy as jnp
from jax.experimental import pallas as pl
from jax.experimental.pallas import tpu as pltpu

_N = 4096
_H = 128
_Hh = _H // 2
_R = 2

_BM = 256
_BK = 2048
_GM = _N // _BM
_GK = _N // _BK


def _bigcn_kernel(inps_ref, fw_ref, bw_ref, Wfw_ref, bfw_ref, Wbw_ref,
                  bbw_ref, W1_ref, b1_ref, out_ref, acc_ref, h_ref):
    m = pl.program_id(0)
    k = pl.program_id(1)

    @pl.when(m == 0)
    def _project():
        x = inps_ref[pl.ds(k * _BK, _BK), :]
        for i in range(_R):
            h_ref[pl.ds(k * _BK, _BK), i * _H:i * _H + _Hh] = jnp.dot(
                x, Wbw_ref[i], preferred_element_type=jnp.float32)
            h_ref[pl.ds(k * _BK, _BK), i * _H + _Hh:(i + 1) * _H] = jnp.dot(
                x, Wfw_ref[i], preferred_element_type=jnp.float32)

    hblk = h_ref[pl.ds(k * _BK, _BK), :]
    left = jnp.dot(bw_ref[0], hblk[:, :_Hh],
                   preferred_element_type=jnp.float32)
    right = jnp.dot(fw_ref[0], hblk[:, _Hh:_H],
                    preferred_element_type=jnp.float32)
    for i in range(1, _R):
        left = left + jnp.dot(bw_ref[i], hblk[:, i * _H:i * _H + _Hh],
                              preferred_element_type=jnp.float32)
        right = right + jnp.dot(fw_ref[i], hblk[:, i * _H + _Hh:(i + 1) * _H],
                                preferred_element_type=jnp.float32)
    partial = jnp.concatenate([left, right], axis=1)

    @pl.when(k == 0)
    def _first():
        acc_ref[...] = partial

    @pl.when(k > 0)
    def _accum():
        acc_ref[...] += partial

    @pl.when(k == _GK - 1)
    def _epilogue():
        bias = jnp.concatenate(
            [jnp.sum(bbw_ref[...], axis=0), jnp.sum(bfw_ref[...], axis=0)])
        s = jnp.maximum(acc_ref[...] + bias[None, :], 0.0)
        feats = jax.lax.dot_general(
            s, W1_ref[...], (((1,), (1,)), ((), ())),
            preferred_element_type=jnp.float32)
        out_ref[...] = feats + b1_ref[...][None, :] + \
            inps_ref[pl.ds(m * _BM, _BM), :]


@functools.partial(jax.jit, static_argnames=())
def kernel(inps, fw_adjs, bw_adjs, W_fw, b_fw, W_bw, b_bw, W1, b1):
    return pl.pallas_call(
        _bigcn_kernel,
        grid=(_GM, _GK),
        in_specs=[
            pl.BlockSpec((_N, _H), lambda m, k: (0, 0)),
            pl.BlockSpec((_R, _BM, _BK), lambda m, k: (0, m, k)),
            pl.BlockSpec((_R, _BM, _BK), lambda m, k: (0, m, k)),
            pl.BlockSpec((_R, _H, _Hh), lambda m, k: (0, 0, 0)),
            pl.BlockSpec((_R, _Hh), lambda m, k: (0, 0)),
            pl.BlockSpec((_R, _H, _Hh), lambda m, k: (0, 0, 0)),
            pl.BlockSpec((_R, _Hh), lambda m, k: (0, 0)),
            pl.BlockSpec((_H, _H), lambda m, k: (0, 0)),
            pl.BlockSpec((_H,), lambda m, k: (0,)),
        ],
        out_specs=pl.BlockSpec((_BM, _H), lambda m, k: (m, 0)),
        out_shape=jax.ShapeDtypeStruct((_N, _H), jnp.float32),
        scratch_shapes=[
            pltpu.VMEM((_BM, _H), jnp.float32),
            pltpu.VMEM((_N, _R * _H), jnp.float32),
        ],
    )(inps, fw_adjs, bw_adjs, W_fw, b_fw, W_bw, b_bw, W1, b1)


# speedup vs baseline: 1.0245x; 1.0245x over previous
import functools

import jax
import jax.numpy as jnp
from jax.experimental import pallas as pl
from jax.experimental.pallas import tpu as pltpu

_N = 4096
_H = 128
_Hh = _H // 2
_R = 2

_BM = 256
_GM = _N // _BM
_BH = _BM // 2


def _bigcn_kernel(inps_ref, fw_ref, bw_ref, Wfw_ref, bfw_ref, Wbw_ref,
                  bbw_ref, W1_ref, b1_ref, out_ref, h_ref):
    m = pl.program_id(0)

    @pl.when(m == 0)
    def _project():
        x = inps_ref[...]
        for i in range(_R):
            h_ref[:, i * _H:i * _H + _Hh] = jnp.dot(
                x, Wbw_ref[i], preferred_element_type=jnp.float32)
            h_ref[:, i * _H + _Hh:(i + 1) * _H] = jnp.dot(
                x, Wfw_ref[i], preferred_element_type=jnp.float32)

    bias = jnp.concatenate(
        [jnp.sum(bbw_ref[...], axis=0), jnp.sum(bfw_ref[...], axis=0)])

    for hrow in range(2):
        rows = pl.ds(hrow * _BH, _BH)
        left = jnp.dot(bw_ref[0, rows, :], h_ref[:, :_Hh],
                       preferred_element_type=jnp.float32)
        right = jnp.dot(fw_ref[0, rows, :], h_ref[:, _Hh:_H],
                        preferred_element_type=jnp.float32)
        for i in range(1, _R):
            left = left + jnp.dot(bw_ref[i, rows, :],
                                  h_ref[:, i * _H:i * _H + _Hh],
                                  preferred_element_type=jnp.float32)
            right = right + jnp.dot(fw_ref[i, rows, :],
                                    h_ref[:, i * _H + _Hh:(i + 1) * _H],
                                    preferred_element_type=jnp.float32)

        s = jnp.maximum(
            jnp.concatenate([left, right], axis=1) + bias[None, :], 0.0)
        feats = jax.lax.dot_general(
            s, W1_ref[...], (((1,), (1,)), ((), ())),
            preferred_element_type=jnp.float32)
        out_ref[rows, :] = feats + b1_ref[...][None, :] + \
            inps_ref[pl.ds(m * _BM + hrow * _BH, _BH), :]


@functools.partial(jax.jit, static_argnames=())
def kernel(inps, fw_adjs, bw_adjs, W_fw, b_fw, W_bw, b_bw, W1, b1):
    return pl.pallas_call(
        _bigcn_kernel,
        grid=(_GM,),
        in_specs=[
            pl.BlockSpec((_N, _H), lambda m: (0, 0)),
            pl.BlockSpec((_R, _BM, _N), lambda m: (0, m, 0)),
            pl.BlockSpec((_R, _BM, _N), lambda m: (0, m, 0)),
            pl.BlockSpec((_R, _H, _Hh), lambda m: (0, 0, 0)),
            pl.BlockSpec((_R, _Hh), lambda m: (0, 0)),
            pl.BlockSpec((_R, _H, _Hh), lambda m: (0, 0, 0)),
            pl.BlockSpec((_R, _Hh), lambda m: (0, 0)),
            pl.BlockSpec((_H, _H), lambda m: (0, 0)),
            pl.BlockSpec((_H,), lambda m: (0,)),
        ],
        out_specs=pl.BlockSpec((_BM, _H), lambda m: (m, 0)),
        out_shape=jax.ShapeDtypeStruct((_N, _H), jnp.float32),
        scratch_shapes=[pltpu.VMEM((_N, _R * _H), jnp.float32)],
    )(inps, fw_adjs, bw_adjs, W_fw, b_fw, W_bw, b_bw, W1, b1)
